# baseline (speedup 1.0000x reference)
import functools

import jax
import jax.numpy as jnp
from jax import lax
from jax.experimental import pallas as pl
from jax.experimental.pallas import tpu as pltpu
from jax.experimental.pallas import tpu_sc as plsc

VOCAB = 100000
D = 128
BATCH = 4096
SEQ = 200

NC = 2
NS = 16
L = 16
NW = NC * NS
BPW = BATCH // NW
CA = 104
CB = SEQ - CA
NB = D // 32


def _sc_kernel(pidx_hbm, hidx_hbm, table_hbm, wb_hbm, out_hbm,
               idx_v, rows_v, wb_v, logp_v, logh_v, outv_v, sem0, sem1):
    wid = lax.axis_index("s") * NC + lax.axis_index("c")
    base = wid * BPW
    sems = (sem0, sem1)
    lane = lax.iota(jnp.int32, L)
    neg_inf = jnp.full((L,), -jnp.inf, jnp.float32)
    NWRD = D // 32

    pltpu.sync_copy(wb_hbm, wb_v)

    def copies(i, b):
        return (
            pltpu.make_async_copy(table_hbm.at[idx_v.at[i, pl.ds(0, CA)]],
                                  rows_v.at[b, pl.ds(0, CA)], sems[b]),
            pltpu.make_async_copy(table_hbm.at[idx_v.at[i, pl.ds(CA, CB)]],
                                  rows_v.at[b, pl.ds(CA, CB)], sems[b]),
        )

    def issue(i, b):
        for cp in copies(i, b):
            cp.start()

    def wait(i, b):
        for cp in copies(i, b):
            cp.wait()

    def run_phase(side_hbm, w_off, log_ref):
        pltpu.sync_copy(side_hbm.at[wid], idx_v)
        wlo = [wb_v[pl.ds(w_off + c * 32, L)] for c in range(NWRD)]
        whi = [wb_v[pl.ds(w_off + c * 32 + L, L)] for c in range(NWRD)]

        issue(0, 0)

        def pair_body(i2, _):
            for b in (0, 1):
                i = 2 * i2 + b

                @pl.when(i + 1 < BPW)
                def _():
                    issue(i + 1, 1 - b)

                wait(i, b)

                def row_body(j, carry):
                    lo, hi = carry
                    u = [rows_v[b, j, pl.ds(c * L, L)] for c in range(NWRD)]
                    lo = tuple(
                        jnp.maximum(lo[c],
                                    plsc.bitcast(u[c] << 16, jnp.float32))
                        for c in range(NWRD))
                    hi = tuple(
                        jnp.maximum(hi[c], plsc.bitcast(u[c], jnp.float32))
                        for c in range(NWRD))
                    return lo, hi

                init = ((neg_inf,) * NWRD, (neg_inf,) * NWRD)
                mlo, mhi = lax.fori_loop(0, SEQ, row_body, init, unroll=8)
                acc = jnp.zeros((L,), jnp.float32)
                for c in range(NWRD):
                    acc = acc + mlo[c] * wlo[c] + mhi[c] * whi[c]
                for sh in (1, 2, 4, 8):
                    acc = acc + acc.at[lane ^ sh].get(
                        mode="promise_in_bounds")
                log_ref[i, :] = acc
            return 0

        lax.fori_loop(0, BPW // 2, pair_body, 0)

    run_phase(pidx_hbm, 0, logp_v)
    run_phase(hidx_hbm, D, logh_v)

    bvec = wb_v[pl.ds(2 * D, L)]
    for g in range(BPW // L):
        x = bvec
        for k in range(L):
            r = g * L + k
            x = jnp.where(lane == k,
                          x + logp_v[r, pl.ds(0, L)] + logh_v[r, pl.ds(0, L)],
                          x)
        outv_v[pl.ds(g * L, L)] = 1.0 / (1.0 + jnp.exp(-x))
    pltpu.sync_copy(outv_v, out_hbm.at[pl.ds(base, BPW)])


PACK_BLK = 1000


def _pack_kernel(t_ref, o_ref):
    u = lax.bitcast_convert_type(t_ref[...], jnp.uint32)
    r = u + jnp.uint32(0x7FFF) + ((u >> 16) & jnp.uint32(1))
    h = r >> 16
    o_ref[...] = lax.bitcast_convert_type(h[:, :64] | (h[:, 64:] << 16),
                                          jnp.int32)


def _pack_table(table):
    return pl.pallas_call(
        _pack_kernel,
        grid=(VOCAB // PACK_BLK,),
        in_specs=[pl.BlockSpec((PACK_BLK, D), lambda i: (i, 0))],
        out_specs=pl.BlockSpec((PACK_BLK, D // 2), lambda i: (i, 0)),
        out_shape=jax.ShapeDtypeStruct((VOCAB, D // 2), jnp.int32),
    )(table)


@jax.jit
def kernel(premise, hypothesis, table, W, b):
    pidx = premise.reshape(NW, BPW, SEQ)
    hidx = hypothesis.reshape(NW, BPW, SEQ)
    tablei = _pack_table(table)
    ws = W.reshape(2, 2, D // 32, L)
    wperm = ws.transpose(0, 2, 1, 3).reshape(-1)
    wb = jnp.concatenate([wperm, jnp.broadcast_to(b, (L,))])

    mesh = plsc.VectorSubcoreMesh(core_axis_name="c", subcore_axis_name="s")
    f = pl.kernel(
        _sc_kernel,
        mesh=mesh,
        compiler_params=pltpu.CompilerParams(needs_layout_passes=False,
                                             use_tc_tiling_on_sc=False),
        out_type=jax.ShapeDtypeStruct((BATCH,), jnp.float32),
        scratch_types=[
            pltpu.VMEM((BPW, SEQ), jnp.int32),
            pltpu.VMEM((2, SEQ, D // 2), jnp.int32),
            pltpu.VMEM((2 * D + L,), jnp.float32),
            pltpu.VMEM((BPW, L), jnp.float32),
            pltpu.VMEM((BPW, L), jnp.float32),
            pltpu.VMEM((BPW,), jnp.float32),
            pltpu.SemaphoreType.DMA,
            pltpu.SemaphoreType.DMA,
        ],
    )
    return f(pidx, hidx, tablei, wb)


# speedup vs baseline: 1.1089x; 1.1089x over previous
import functools

import jax
import jax.numpy as jnp
from jax import lax
from jax.experimental import pallas as pl
from jax.experimental.pallas import tpu as pltpu
from jax.experimental.pallas import tpu_sc as plsc

VOCAB = 100000
D = 128
BATCH = 4096
SEQ = 200

NC = 2
NS = 16
L = 16
NW = NC * NS
BPW = BATCH // NW
CA = 104
CB = SEQ - CA
NB = D // 32


def _sc_kernel(pidx_hbm, hidx_hbm, table_hbm, wb_hbm, out_hbm,
               idx_v, rows_v, wb_v, logp_v, logh_v, outv_v, sem0, sem1):
    wid = lax.axis_index("s") * NC + lax.axis_index("c")
    base = wid * BPW
    sems = (sem0, sem1)
    lane = lax.iota(jnp.int32, L)
    neg_inf = jnp.full((L,), -jnp.inf, jnp.float32)
    NWRD = D // 32

    pltpu.sync_copy(wb_hbm, wb_v)

    def copies(i, b):
        return (
            pltpu.make_async_copy(table_hbm.at[idx_v.at[i, pl.ds(0, CA)]],
                                  rows_v.at[b, pl.ds(0, CA)], sems[b]),
            pltpu.make_async_copy(table_hbm.at[idx_v.at[i, pl.ds(CA, CB)]],
                                  rows_v.at[b, pl.ds(CA, CB)], sems[b]),
        )

    def issue(i, b):
        for cp in copies(i, b):
            cp.start()

    def wait(i, b):
        for cp in copies(i, b):
            cp.wait()

    def run_phase(side_hbm, w_off, log_ref):
        pltpu.sync_copy(side_hbm.at[wid], idx_v)
        wlo = [wb_v[pl.ds(w_off + c * 32, L)] for c in range(NWRD)]
        whi = [wb_v[pl.ds(w_off + c * 32 + L, L)] for c in range(NWRD)]

        issue(0, 0)

        def pair_body(i2, _):
            for b in (0, 1):
                i = 2 * i2 + b

                @pl.when(i + 1 < BPW)
                def _():
                    issue(i + 1, 1 - b)

                wait(i, b)

                def row_body(j, carry):
                    lo, hi = carry
                    u = [rows_v[b, j, pl.ds(c * L, L)] for c in range(NWRD)]
                    lo = tuple(
                        jnp.maximum(lo[c],
                                    plsc.bitcast(u[c] << 16, jnp.float32))
                        for c in range(NWRD))
                    hi = tuple(
                        jnp.maximum(hi[c], plsc.bitcast(u[c], jnp.float32))
                        for c in range(NWRD))
                    return lo, hi

                init = ((neg_inf,) * NWRD, (neg_inf,) * NWRD)
                mlo, mhi = lax.fori_loop(0, SEQ, row_body, init, unroll=8)
                acc = jnp.zeros((L,), jnp.float32)
                for c in range(NWRD):
                    acc = acc + mlo[c] * wlo[c] + mhi[c] * whi[c]
                for sh in (1, 2, 4, 8):
                    acc = acc + acc.at[lane ^ sh].get(
                        mode="promise_in_bounds")
                log_ref[i, :] = acc
            return 0

        lax.fori_loop(0, BPW // 2, pair_body, 0)

    run_phase(pidx_hbm, 0, logp_v)
    run_phase(hidx_hbm, D, logh_v)

    bvec = wb_v[pl.ds(2 * D, L)]
    for g in range(BPW // L):
        x = bvec
        for k in range(L):
            r = g * L + k
            x = jnp.where(lane == k,
                          x + logp_v[r, pl.ds(0, L)] + logh_v[r, pl.ds(0, L)],
                          x)
        outv_v[pl.ds(g * L, L)] = 1.0 / (1.0 + jnp.exp(-x))
    pltpu.sync_copy(outv_v, out_hbm.at[pl.ds(base, BPW)])


PACK_BLK = 4000


def _pack_kernel(t_ref, o_ref):
    u = lax.bitcast_convert_type(t_ref[...], jnp.uint32)
    r = u + jnp.uint32(0x7FFF) + ((u >> 16) & jnp.uint32(1))
    h = r >> 16
    o_ref[...] = lax.bitcast_convert_type(h[:, :64] | (h[:, 64:] << 16),
                                          jnp.int32)


def _pack_table(table):
    return pl.pallas_call(
        _pack_kernel,
        grid=(VOCAB // PACK_BLK,),
        in_specs=[pl.BlockSpec((PACK_BLK, D), lambda i: (i, 0))],
        out_specs=pl.BlockSpec((PACK_BLK, D // 2), lambda i: (i, 0)),
        out_shape=jax.ShapeDtypeStruct((VOCAB, D // 2), jnp.int32),
    )(table)


@jax.jit
def kernel(premise, hypothesis, table, W, b):
    pidx = premise.reshape(NW, BPW, SEQ)
    hidx = hypothesis.reshape(NW, BPW, SEQ)
    tablei = _pack_table(table)
    ws = W.reshape(2, 2, D // 32, L)
    wperm = ws.transpose(0, 2, 1, 3).reshape(-1)
    wb = jnp.concatenate([wperm, jnp.broadcast_to(b, (L,))])

    mesh = plsc.VectorSubcoreMesh(core_axis_name="c", subcore_axis_name="s")
    f = pl.kernel(
        _sc_kernel,
        mesh=mesh,
        compiler_params=pltpu.CompilerParams(needs_layout_passes=False,
                                             use_tc_tiling_on_sc=False),
        out_type=jax.ShapeDtypeStruct((BATCH,), jnp.float32),
        scratch_types=[
            pltpu.VMEM((BPW, SEQ), jnp.int32),
            pltpu.VMEM((2, SEQ, D // 2), jnp.int32),
            pltpu.VMEM((2 * D + L,), jnp.float32),
            pltpu.VMEM((BPW, L), jnp.float32),
            pltpu.VMEM((BPW, L), jnp.float32),
            pltpu.VMEM((BPW,), jnp.float32),
            pltpu.SemaphoreType.DMA,
            pltpu.SemaphoreType.DMA,
        ],
    )
    return f(pidx, hidx, tablei, wb)


# speedup vs baseline: 1.1276x; 1.0168x over previous
import functools

import jax
import jax.numpy as jnp
from jax import lax
from jax.experimental import pallas as pl
from jax.experimental.pallas import tpu as pltpu
from jax.experimental.pallas import tpu_sc as plsc

VOCAB = 100000
D = 128
BATCH = 4096
SEQ = 200

NC = 2
NS = 16
L = 16
NW = NC * NS
BPW = BATCH // NW
CA = 104
CB = SEQ - CA
NB = D // 32


def _sc_kernel(pidx_hbm, hidx_hbm, table_hbm, wb_hbm, out_hbm,
               idx_v, rows_v, wb_v, logp_v, logh_v, outv_v, sem0, sem1):
    wid = lax.axis_index("s") * NC + lax.axis_index("c")
    base = wid * BPW
    sems = (sem0, sem1)
    lane = lax.iota(jnp.int32, L)
    neg_inf = jnp.full((L,), -jnp.inf, jnp.float32)
    NWRD = D // 32

    pltpu.sync_copy(wb_hbm, wb_v)

    def copies(i, b):
        return (
            pltpu.make_async_copy(table_hbm.at[idx_v.at[i, pl.ds(0, CA)]],
                                  rows_v.at[b, pl.ds(0, CA)], sems[b]),
            pltpu.make_async_copy(table_hbm.at[idx_v.at[i, pl.ds(CA, CB)]],
                                  rows_v.at[b, pl.ds(CA, CB)], sems[b]),
        )

    def issue(i, b):
        for cp in copies(i, b):
            cp.start()

    def wait(i, b):
        for cp in copies(i, b):
            cp.wait()

    def run_phase(side_hbm, w_off, log_ref):
        pltpu.sync_copy(side_hbm.at[wid], idx_v)
        wlo = [wb_v[pl.ds(w_off + c * 32, L)] for c in range(NWRD)]
        whi = [wb_v[pl.ds(w_off + c * 32 + L, L)] for c in range(NWRD)]

        issue(0, 0)

        def pair_body(i2, _):
            for b in (0, 1):
                i = 2 * i2 + b

                @pl.when(i + 1 < BPW)
                def _():
                    issue(i + 1, 1 - b)

                wait(i, b)

                def row_body(j, carry):
                    lo, hi = carry
                    u = [rows_v[b, j, pl.ds(c * L, L)] for c in range(NWRD)]
                    lo = tuple(
                        jnp.maximum(lo[c],
                                    plsc.bitcast(u[c] << 16, jnp.float32))
                        for c in range(NWRD))
                    hi = tuple(
                        jnp.maximum(hi[c], plsc.bitcast(u[c], jnp.float32))
                        for c in range(NWRD))
                    return lo, hi

                init = ((neg_inf,) * NWRD, (neg_inf,) * NWRD)
                mlo, mhi = lax.fori_loop(0, SEQ, row_body, init, unroll=8)
                acc = jnp.zeros((L,), jnp.float32)
                for c in range(NWRD):
                    acc = acc + mlo[c] * wlo[c] + mhi[c] * whi[c]
                for sh in (1, 2, 4, 8):
                    acc = acc + acc.at[lane ^ sh].get(
                        mode="promise_in_bounds")
                log_ref[i, :] = acc
            return 0

        lax.fori_loop(0, BPW // 2, pair_body, 0)

    run_phase(pidx_hbm, 0, logp_v)
    run_phase(hidx_hbm, D, logh_v)

    bvec = wb_v[pl.ds(2 * D, L)]
    for g in range(BPW // L):
        x = bvec
        for k in range(L):
            r = g * L + k
            x = jnp.where(lane == k,
                          x + logp_v[r, pl.ds(0, L)] + logh_v[r, pl.ds(0, L)],
                          x)
        outv_v[pl.ds(g * L, L)] = 1.0 / (1.0 + jnp.exp(-x))
    pltpu.sync_copy(outv_v, out_hbm.at[pl.ds(base, BPW)])


PACK_BLK = 10000


def _pack_kernel(t_ref, o_ref):
    u = lax.bitcast_convert_type(t_ref[...], jnp.uint32)
    r = u + jnp.uint32(0x7FFF) + ((u >> 16) & jnp.uint32(1))
    h = r >> 16
    o_ref[...] = lax.bitcast_convert_type(h[:, :64] | (h[:, 64:] << 16),
                                          jnp.int32)


def _pack_table(table):
    return pl.pallas_call(
        _pack_kernel,
        grid=(VOCAB // PACK_BLK,),
        in_specs=[pl.BlockSpec((PACK_BLK, D), lambda i: (i, 0))],
        out_specs=pl.BlockSpec((PACK_BLK, D // 2), lambda i: (i, 0)),
        out_shape=jax.ShapeDtypeStruct((VOCAB, D // 2), jnp.int32),
    )(table)


@jax.jit
def kernel(premise, hypothesis, table, W, b):
    pidx = premise.reshape(NW, BPW, SEQ)
    hidx = hypothesis.reshape(NW, BPW, SEQ)
    tablei = _pack_table(table)
    ws = W.reshape(2, 2, D // 32, L)
    wperm = ws.transpose(0, 2, 1, 3).reshape(-1)
    wb = jnp.concatenate([wperm, jnp.broadcast_to(b, (L,))])

    mesh = plsc.VectorSubcoreMesh(core_axis_name="c", subcore_axis_name="s")
    f = pl.kernel(
        _sc_kernel,
        mesh=mesh,
        compiler_params=pltpu.CompilerParams(needs_layout_passes=False,
                                             use_tc_tiling_on_sc=False),
        out_type=jax.ShapeDtypeStruct((BATCH,), jnp.float32),
        scratch_types=[
            pltpu.VMEM((BPW, SEQ), jnp.int32),
            pltpu.VMEM((2, SEQ, D // 2), jnp.int32),
            pltpu.VMEM((2 * D + L,), jnp.float32),
            pltpu.VMEM((BPW, L), jnp.float32),
            pltpu.VMEM((BPW, L), jnp.float32),
            pltpu.VMEM((BPW,), jnp.float32),
            pltpu.SemaphoreType.DMA,
            pltpu.SemaphoreType.DMA,
        ],
    )
    return f(pidx, hidx, tablei, wb)
